# baseline (speedup 1.0000x reference)
import jax
import jax.numpy as jnp
from jax import lax
from jax.experimental import pallas as pl
from jax.experimental.pallas import tpu as pltpu
from jax.experimental.pallas import tpu_sc as plsc

_B = 1024
_W = 200
_D = 64
_N = _B * _W
_NC = 2
_NS = 16
_NW = _NC * _NS
_CHUNK = 128
_NCHUNKS = _N // _CHUNK
_CPW = _NCHUNKS // _NW


def _emb_body(idx_hbm, table_hbm, out_hbm, idx_v, rows_v, gsem):
    wid = lax.axis_index("s") * _NC + lax.axis_index("c")
    pltpu.sync_copy(idx_hbm.at[wid], idx_v)

    def body(j, carry):
        pltpu.async_copy(table_hbm.at[idx_v.at[j]], rows_v, gsem).wait()
        pltpu.sync_copy(rows_v, out_hbm.at[wid, j])
        return carry

    lax.fori_loop(0, _CPW, body, 0)


def kernel(token_ids, table):
    idx = token_ids.reshape(_NW, _CPW, _CHUNK).astype(jnp.int32)
    mesh = plsc.VectorSubcoreMesh(core_axis_name="c", subcore_axis_name="s")
    out = pl.kernel(
        _emb_body,
        out_type=jax.ShapeDtypeStruct((_NW, _CPW, _CHUNK, _D), jnp.float32),
        mesh=mesh,
        scratch_types=[
            pltpu.VMEM((_CPW, _CHUNK), jnp.int32),
            pltpu.VMEM((_CHUNK, _D), jnp.float32),
            pltpu.SemaphoreType.DMA,
        ],
        compiler_params=pltpu.CompilerParams(use_tc_tiling_on_sc=False),
    )(idx, table)
    return out.reshape(_B, _W, _D)


# speedup vs baseline: 1.0476x; 1.0476x over previous
import jax
import jax.numpy as jnp
from jax import lax
from jax.experimental import pallas as pl
from jax.experimental.pallas import tpu as pltpu
from jax.experimental.pallas import tpu_sc as plsc

_B = 1024
_W = 200
_D = 64
_N = _B * _W
_NC = 2
_NS = 16
_NW = _NC * _NS
_CHUNK = 128
_NCHUNKS = _N // _CHUNK
_CPW = _NCHUNKS // _NW


_NBUF = 5
_NGRP = _CPW // _NBUF


def _emb_body(idx_hbm, table_hbm, out_hbm, idx_v, rows_v, gsem, ssem):
    wid = lax.axis_index("s") * _NC + lax.axis_index("c")
    pltpu.sync_copy(idx_hbm.at[wid], idx_v)

    @pl.loop(0, _NGRP)
    def _grp(g):
        for k in range(_NBUF):
            @pl.when(g > 0)
            def _():
                pltpu.make_async_copy(
                    rows_v.at[k], out_hbm.at[wid, 0], ssem.at[k]
                ).wait()
            pltpu.async_copy(
                table_hbm.at[idx_v.at[g * _NBUF + k]], rows_v.at[k], gsem.at[k]
            )
        for k in range(_NBUF):
            pltpu.make_async_copy(
                table_hbm.at[idx_v.at[0]], rows_v.at[k], gsem.at[k]
            ).wait()
            pltpu.async_copy(rows_v.at[k], out_hbm.at[wid, g * _NBUF + k],
                             ssem.at[k])

    for k in range(_NBUF):
        pltpu.make_async_copy(
            rows_v.at[k], out_hbm.at[wid, 0], ssem.at[k]
        ).wait()


def kernel(token_ids, table):
    idx = token_ids.reshape(_NW, _CPW, _CHUNK).astype(jnp.int32)
    mesh = plsc.VectorSubcoreMesh(core_axis_name="c", subcore_axis_name="s")
    out = pl.kernel(
        _emb_body,
        out_type=jax.ShapeDtypeStruct((_NW, _CPW, _CHUNK, _D), jnp.float32),
        mesh=mesh,
        scratch_types=[
            pltpu.VMEM((_CPW, _CHUNK), jnp.int32),
            pltpu.VMEM((_NBUF, _CHUNK, _D), jnp.float32),
            pltpu.SemaphoreType.DMA((_NBUF,)),
            pltpu.SemaphoreType.DMA((_NBUF,)),
        ],
        compiler_params=pltpu.CompilerParams(use_tc_tiling_on_sc=False),
    )(idx, table)
    return out.reshape(_B, _W, _D)
